# Initial kernel scaffold; baseline (speedup 1.0000x reference)
import jax
import jax.numpy as jnp
from jax.experimental import pallas as pl


def kernel(U, Wq, bq, Wk, bk):
    raise NotImplementedError("write your pallas kernel here")


# speedup vs baseline: 2.0527x; 2.0527x over previous
import functools
import math

import jax
import jax.numpy as jnp
from jax.experimental import pallas as pl
from jax.experimental.pallas import tpu as pltpu

N, T, IN_DIM, KEY_DIM, TOPK = 8192, 12, 128, 64, 8

_POOL_BLOCK = 1024
_SCORE_BLOCK = 512


def _pool_proj_kernel(u_ref, wq_ref, bq_ref, wk_ref, bk_ref, q_ref, k_ref):
    acc = u_ref[:, 0:IN_DIM]
    for t in range(1, T):
        acc = acc + u_ref[:, t * IN_DIM:(t + 1) * IN_DIM]
    pooled = acc * (1.0 / T)
    dn = (((1,), (1,)), ((), ()))
    q_ref[...] = jax.lax.dot_general(
        pooled, wq_ref[...], dn, preferred_element_type=jnp.float32) + bq_ref[...]
    k_ref[...] = jax.lax.dot_general(
        pooled, wk_ref[...], dn, preferred_element_type=jnp.float32) + bk_ref[...]


def _topk_kernel(q_ref, k_ref, vals_ref, cols_ref, *, n, topk):
    scale = 1.0 / math.sqrt(KEY_DIM)
    dn = (((1,), (1,)), ((), ()))
    s = jax.lax.dot_general(
        q_ref[...], k_ref[...], dn, preferred_element_type=jnp.float32) * scale
    iota = jax.lax.broadcasted_iota(jnp.int32, s.shape, 1)
    m0 = jnp.max(s, axis=1, keepdims=True)
    denom = jnp.sum(jnp.exp(s - m0), axis=1, keepdims=True)

    vals = []
    cols = []
    neg_inf = jnp.float32(-jnp.inf)
    for _ in range(topk):
        m = jnp.max(s, axis=1, keepdims=True)
        idx = jnp.min(jnp.where(s == m, iota, n), axis=1, keepdims=True)
        vals.append(m)
        cols.append(idx)
        s = jnp.where(iota == idx, neg_inf, s)
    vals8 = jnp.concatenate(vals, axis=1)
    cols8 = jnp.concatenate(cols, axis=1)

    attn8 = jnp.exp(vals8 - m0) / denom

    out_v = []
    out_c = []
    active = jnp.ones(cols8.shape, dtype=jnp.bool_)
    for _ in range(topk):
        c = jnp.min(jnp.where(active, cols8, n), axis=1, keepdims=True)
        hit = cols8 == c
        v = jnp.sum(jnp.where(hit, attn8, 0.0), axis=1, keepdims=True)
        active = active & ~hit
        out_c.append(c)
        out_v.append(v)
    vals_ref[...] = jnp.concatenate(out_v, axis=1)
    cols_ref[...] = jnp.concatenate(out_c, axis=1)


def kernel(U, Wq, bq, Wk, bk):
    n = U.shape[0]
    u2d = U.reshape(n, T * IN_DIM)
    bq2 = bq.reshape(1, KEY_DIM)
    bk2 = bk.reshape(1, KEY_DIM)

    br1 = _POOL_BLOCK
    q, k = pl.pallas_call(
        _pool_proj_kernel,
        grid=(n // br1,),
        in_specs=[
            pl.BlockSpec((br1, T * IN_DIM), lambda i: (i, 0)),
            pl.BlockSpec((KEY_DIM, IN_DIM), lambda i: (0, 0)),
            pl.BlockSpec((1, KEY_DIM), lambda i: (0, 0)),
            pl.BlockSpec((KEY_DIM, IN_DIM), lambda i: (0, 0)),
            pl.BlockSpec((1, KEY_DIM), lambda i: (0, 0)),
        ],
        out_specs=[
            pl.BlockSpec((br1, KEY_DIM), lambda i: (i, 0)),
            pl.BlockSpec((br1, KEY_DIM), lambda i: (i, 0)),
        ],
        out_shape=[
            jax.ShapeDtypeStruct((n, KEY_DIM), jnp.float32),
            jax.ShapeDtypeStruct((n, KEY_DIM), jnp.float32),
        ],
    )(u2d, Wq, bq2, Wk, bk2)

    br2 = _SCORE_BLOCK
    vals, cols = pl.pallas_call(
        functools.partial(_topk_kernel, n=n, topk=TOPK),
        grid=(n // br2,),
        in_specs=[
            pl.BlockSpec((br2, KEY_DIM), lambda i: (i, 0)),
            pl.BlockSpec((n, KEY_DIM), lambda i: (0, 0)),
        ],
        out_specs=[
            pl.BlockSpec((br2, TOPK), lambda i: (i, 0)),
            pl.BlockSpec((br2, TOPK), lambda i: (i, 0)),
        ],
        out_shape=[
            jax.ShapeDtypeStruct((n, TOPK), jnp.float32),
            jax.ShapeDtypeStruct((n, TOPK), jnp.int32),
        ],
    )(q, k)

    rows = jnp.repeat(jnp.arange(n, dtype=jnp.int32), TOPK)
    indices = jnp.stack([rows, cols.reshape(-1)], axis=0)
    values = vals.reshape(-1)
    return indices, values


# speedup vs baseline: 2.8968x; 1.4112x over previous
import functools
import math

import jax
import jax.numpy as jnp
from jax.experimental import pallas as pl
from jax.experimental.pallas import tpu as pltpu

N, T, IN_DIM, KEY_DIM, TOPK = 8192, 12, 128, 64, 8

_POOL_BLOCK = 1024
_SCORE_BLOCK = 256


def _pool_proj_kernel(u_ref, wq_ref, bq_ref, wk_ref, bk_ref, q_ref, k_ref):
    acc = u_ref[:, 0:IN_DIM]
    for t in range(1, T):
        acc = acc + u_ref[:, t * IN_DIM:(t + 1) * IN_DIM]
    pooled = acc * (1.0 / T)
    dn = (((1,), (1,)), ((), ()))
    q_ref[...] = jax.lax.dot_general(
        pooled, wq_ref[...], dn, preferred_element_type=jnp.float32) + bq_ref[...]
    k_ref[...] = jax.lax.dot_general(
        pooled, wk_ref[...], dn, preferred_element_type=jnp.float32) + bk_ref[...]


def _topk_kernel(q_ref, k_ref, vals_ref, cols_ref, *, n, topk):
    scale = 1.0 / math.sqrt(KEY_DIM)
    dn = (((1,), (1,)), ((), ()))
    s = jax.lax.dot_general(
        q_ref[...], k_ref[...], dn, preferred_element_type=jnp.float32) * scale

    w = n // topk
    br = s.shape[0]
    qbase = jax.lax.broadcasted_iota(jnp.int32, (br, w), 1)
    P = [s[:, k * w:(k + 1) * w] for k in range(topk)]
    Q = [qbase + (k * w) for k in range(topk)]

    def ce(i, j):
        ge = P[i] >= P[j]
        pi = jnp.where(ge, P[i], P[j])
        pj = jnp.where(ge, P[j], P[i])
        qi = jnp.where(ge, Q[i], Q[j])
        qj = jnp.where(ge, Q[j], Q[i])
        P[i], P[j], Q[i], Q[j] = pi, pj, qi, qj

    for i, j in [(0, 1), (2, 3), (4, 5), (6, 7),
                 (0, 2), (1, 3), (4, 6), (5, 7),
                 (1, 2), (5, 6),
                 (0, 4), (1, 5), (2, 6), (3, 7),
                 (2, 4), (3, 5),
                 (1, 2), (3, 4), (5, 6)]:
        ce(i, j)

    vals = []
    cols = []
    for t in range(topk):
        m = jnp.max(P[0], axis=1, keepdims=True)
        idx = jnp.min(jnp.where(P[0] == m, Q[0], n), axis=1, keepdims=True)
        vals.append(m)
        cols.append(idx)
        if t < topk - 1:
            hit = Q[0] == idx
            for c in range(topk - 1 - t):
                P[c] = jnp.where(hit, P[c + 1], P[c])
                Q[c] = jnp.where(hit, Q[c + 1], Q[c])
    vals8 = jnp.concatenate(vals, axis=1)
    cols8 = jnp.concatenate(cols, axis=1)

    m0 = vals[0]
    denom = jnp.sum(jnp.exp(s - m0), axis=1, keepdims=True)

    attn8 = jnp.exp(vals8 - m0) / denom

    out_v = []
    out_c = []
    active = jnp.ones(cols8.shape, dtype=jnp.bool_)
    for _ in range(topk):
        c = jnp.min(jnp.where(active, cols8, n), axis=1, keepdims=True)
        hit = cols8 == c
        v = jnp.sum(jnp.where(hit, attn8, 0.0), axis=1, keepdims=True)
        active = active & ~hit
        out_c.append(c)
        out_v.append(v)
    vals_ref[...] = jnp.concatenate(out_v, axis=1)
    cols_ref[...] = jnp.concatenate(out_c, axis=1)


def kernel(U, Wq, bq, Wk, bk):
    n = U.shape[0]
    u2d = U.reshape(n, T * IN_DIM)
    bq2 = bq.reshape(1, KEY_DIM)
    bk2 = bk.reshape(1, KEY_DIM)

    br1 = _POOL_BLOCK
    q, k = pl.pallas_call(
        _pool_proj_kernel,
        grid=(n // br1,),
        in_specs=[
            pl.BlockSpec((br1, T * IN_DIM), lambda i: (i, 0)),
            pl.BlockSpec((KEY_DIM, IN_DIM), lambda i: (0, 0)),
            pl.BlockSpec((1, KEY_DIM), lambda i: (0, 0)),
            pl.BlockSpec((KEY_DIM, IN_DIM), lambda i: (0, 0)),
            pl.BlockSpec((1, KEY_DIM), lambda i: (0, 0)),
        ],
        out_specs=[
            pl.BlockSpec((br1, KEY_DIM), lambda i: (i, 0)),
            pl.BlockSpec((br1, KEY_DIM), lambda i: (i, 0)),
        ],
        out_shape=[
            jax.ShapeDtypeStruct((n, KEY_DIM), jnp.float32),
            jax.ShapeDtypeStruct((n, KEY_DIM), jnp.float32),
        ],
    )(u2d, Wq, bq2, Wk, bk2)

    br2 = _SCORE_BLOCK
    vals, cols = pl.pallas_call(
        functools.partial(_topk_kernel, n=n, topk=TOPK),
        grid=(n // br2,),
        in_specs=[
            pl.BlockSpec((br2, KEY_DIM), lambda i: (i, 0)),
            pl.BlockSpec((n, KEY_DIM), lambda i: (0, 0)),
        ],
        out_specs=[
            pl.BlockSpec((br2, TOPK), lambda i: (i, 0)),
            pl.BlockSpec((br2, TOPK), lambda i: (i, 0)),
        ],
        out_shape=[
            jax.ShapeDtypeStruct((n, TOPK), jnp.float32),
            jax.ShapeDtypeStruct((n, TOPK), jnp.int32),
        ],
    )(q, k)

    rows = jnp.repeat(jnp.arange(n, dtype=jnp.int32), TOPK)
    indices = jnp.stack([rows, cols.reshape(-1)], axis=0)
    values = vals.reshape(-1)
    return indices, values


# speedup vs baseline: 3.0137x; 1.0404x over previous
import functools
import math

import jax
import jax.numpy as jnp
from jax.experimental import pallas as pl
from jax.experimental.pallas import tpu as pltpu

N, T, IN_DIM, KEY_DIM, TOPK = 8192, 12, 128, 64, 8

_POOL_BLOCK = 1024
_SCORE_BLOCK = 512


def _pool_proj_kernel(u_ref, wq_ref, bq_ref, wk_ref, bk_ref, q_ref, k_ref):
    acc = u_ref[:, 0:IN_DIM]
    for t in range(1, T):
        acc = acc + u_ref[:, t * IN_DIM:(t + 1) * IN_DIM]
    pooled = acc * (1.0 / T)
    dn = (((1,), (1,)), ((), ()))
    q_ref[...] = jax.lax.dot_general(
        pooled, wq_ref[...], dn, preferred_element_type=jnp.float32) + bq_ref[...]
    k_ref[...] = jax.lax.dot_general(
        pooled, wk_ref[...], dn, preferred_element_type=jnp.float32) + bk_ref[...]


def _topk_kernel(q_ref, k_ref, vals_ref, cols_ref, *, n, topk):
    scale = 1.0 / math.sqrt(KEY_DIM)
    dn = (((1,), (1,)), ((), ()))
    s = jax.lax.dot_general(
        q_ref[...], k_ref[...], dn, preferred_element_type=jnp.float32) * scale

    w = n // topk
    br = s.shape[0]
    qbase = jax.lax.broadcasted_iota(jnp.int32, (br, w), 1)
    P = [s[:, k * w:(k + 1) * w] for k in range(topk)]
    Q = [qbase + (k * w) for k in range(topk)]

    def ce(i, j):
        ge = P[i] >= P[j]
        pi = jnp.where(ge, P[i], P[j])
        pj = jnp.where(ge, P[j], P[i])
        qi = jnp.where(ge, Q[i], Q[j])
        qj = jnp.where(ge, Q[j], Q[i])
        P[i], P[j], Q[i], Q[j] = pi, pj, qi, qj

    for i, j in [(0, 1), (2, 3), (4, 5), (6, 7),
                 (0, 2), (1, 3), (4, 6), (5, 7),
                 (1, 2), (5, 6),
                 (0, 4), (1, 5), (2, 6), (3, 7),
                 (2, 4), (3, 5),
                 (1, 2), (3, 4), (5, 6)]:
        ce(i, j)

    vals = []
    cols = []
    for t in range(topk):
        m = jnp.max(P[0], axis=1, keepdims=True)
        idx = jnp.min(jnp.where(P[0] == m, Q[0], n), axis=1, keepdims=True)
        vals.append(m)
        cols.append(idx)
        if t < topk - 1:
            hit = Q[0] == idx
            for c in range(topk - 1 - t):
                P[c] = jnp.where(hit, P[c + 1], P[c])
                Q[c] = jnp.where(hit, Q[c + 1], Q[c])
    vals8 = jnp.concatenate(vals, axis=1)
    cols8 = jnp.concatenate(cols, axis=1)

    m0 = vals[0]
    denom = jnp.sum(jnp.exp(s - m0), axis=1, keepdims=True)

    attn8 = jnp.exp(vals8 - m0) / denom

    out_v = []
    out_c = []
    active = jnp.ones(cols8.shape, dtype=jnp.bool_)
    for _ in range(topk):
        c = jnp.min(jnp.where(active, cols8, n), axis=1, keepdims=True)
        hit = cols8 == c
        v = jnp.sum(jnp.where(hit, attn8, 0.0), axis=1, keepdims=True)
        active = active & ~hit
        out_c.append(c)
        out_v.append(v)
    vals_ref[...] = jnp.concatenate(out_v, axis=1)
    cols_ref[...] = jnp.concatenate(out_c, axis=1)


def kernel(U, Wq, bq, Wk, bk):
    n = U.shape[0]
    u2d = U.reshape(n, T * IN_DIM)
    bq2 = bq.reshape(1, KEY_DIM)
    bk2 = bk.reshape(1, KEY_DIM)

    br1 = _POOL_BLOCK
    q, k = pl.pallas_call(
        _pool_proj_kernel,
        grid=(n // br1,),
        in_specs=[
            pl.BlockSpec((br1, T * IN_DIM), lambda i: (i, 0)),
            pl.BlockSpec((KEY_DIM, IN_DIM), lambda i: (0, 0)),
            pl.BlockSpec((1, KEY_DIM), lambda i: (0, 0)),
            pl.BlockSpec((KEY_DIM, IN_DIM), lambda i: (0, 0)),
            pl.BlockSpec((1, KEY_DIM), lambda i: (0, 0)),
        ],
        out_specs=[
            pl.BlockSpec((br1, KEY_DIM), lambda i: (i, 0)),
            pl.BlockSpec((br1, KEY_DIM), lambda i: (i, 0)),
        ],
        out_shape=[
            jax.ShapeDtypeStruct((n, KEY_DIM), jnp.float32),
            jax.ShapeDtypeStruct((n, KEY_DIM), jnp.float32),
        ],
    )(u2d, Wq, bq2, Wk, bk2)

    br2 = _SCORE_BLOCK
    vals, cols = pl.pallas_call(
        functools.partial(_topk_kernel, n=n, topk=TOPK),
        grid=(n // br2,),
        in_specs=[
            pl.BlockSpec((br2, KEY_DIM), lambda i: (i, 0)),
            pl.BlockSpec((n, KEY_DIM), lambda i: (0, 0)),
        ],
        out_specs=[
            pl.BlockSpec((br2, TOPK), lambda i: (i, 0)),
            pl.BlockSpec((br2, TOPK), lambda i: (i, 0)),
        ],
        out_shape=[
            jax.ShapeDtypeStruct((n, TOPK), jnp.float32),
            jax.ShapeDtypeStruct((n, TOPK), jnp.int32),
        ],
    )(q, k)

    rows = jnp.repeat(jnp.arange(n, dtype=jnp.int32), TOPK)
    indices = jnp.stack([rows, cols.reshape(-1)], axis=0)
    values = vals.reshape(-1)
    return indices, values


# speedup vs baseline: 3.0662x; 1.0174x over previous
import functools
import math

import jax
import jax.numpy as jnp
from jax.experimental import pallas as pl
from jax.experimental.pallas import tpu as pltpu
from jax.experimental.pallas import tpu_sc as plsc

N, T, IN_DIM, KEY_DIM, TOPK = 8192, 12, 128, 64, 8

_POOL_BLOCK = 1024
_SCORE_BLOCK = 512


def _pool_proj_kernel(u_ref, wq_ref, bq_ref, wk_ref, bk_ref, q_ref, k_ref):
    acc = u_ref[:, 0:IN_DIM]
    for t in range(1, T):
        acc = acc + u_ref[:, t * IN_DIM:(t + 1) * IN_DIM]
    pooled = acc * (1.0 / T)
    dn = (((1,), (1,)), ((), ()))
    q_ref[...] = jax.lax.dot_general(
        pooled, wq_ref[...], dn, preferred_element_type=jnp.float32) + bq_ref[...]
    k_ref[...] = jax.lax.dot_general(
        pooled, wk_ref[...], dn, preferred_element_type=jnp.float32) + bk_ref[...]


def _topk_kernel(q_ref, k_ref, vals_ref, cols_ref, *, n, topk):
    scale = 1.0 / math.sqrt(KEY_DIM)
    dn = (((1,), (1,)), ((), ()))
    s = jax.lax.dot_general(
        q_ref[...], k_ref[...], dn, preferred_element_type=jnp.float32) * scale

    m0 = jnp.max(s, axis=1, keepdims=True)

    w = n // topk
    br = s.shape[0]
    qbase = jax.lax.broadcasted_iota(jnp.int32, (br, w), 1)
    P = [s[:, k * w:(k + 1) * w] for k in range(topk)]
    Q = [qbase + (k * w) for k in range(topk)]

    denom = jnp.zeros_like(m0)
    for k in range(topk):
        denom = denom + jnp.sum(jnp.exp(P[k] - m0), axis=1, keepdims=True)

    def ce(i, j):
        ge = P[i] >= P[j]
        pi = jnp.where(ge, P[i], P[j])
        pj = jnp.where(ge, P[j], P[i])
        qi = jnp.where(ge, Q[i], Q[j])
        qj = jnp.where(ge, Q[j], Q[i])
        P[i], P[j], Q[i], Q[j] = pi, pj, qi, qj

    for i, j in [(0, 1), (2, 3), (4, 5), (6, 7),
                 (0, 2), (1, 3), (4, 6), (5, 7),
                 (1, 2), (5, 6),
                 (0, 4), (1, 5), (2, 6), (3, 7),
                 (2, 4), (3, 5),
                 (1, 2), (3, 4), (5, 6)]:
        ce(i, j)

    vals = []
    cols = []
    for t in range(topk):
        m = jnp.max(P[0], axis=1, keepdims=True)
        idx = jnp.min(jnp.where(P[0] == m, Q[0], n), axis=1, keepdims=True)
        vals.append(m)
        cols.append(idx)
        if t < topk - 1:
            hit = Q[0] == idx
            for c in range(topk - 1 - t):
                P[c] = jnp.where(hit, P[c + 1], P[c])
                Q[c] = jnp.where(hit, Q[c + 1], Q[c])
    vals8 = jnp.concatenate(vals, axis=1)
    cols8 = jnp.concatenate(cols, axis=1)

    vals_ref[...] = jnp.exp(vals8 - m0) / denom
    cols_ref[...] = cols8


def kernel(U, Wq, bq, Wk, bk):
    n = U.shape[0]
    u2d = U.reshape(n, T * IN_DIM)
    bq2 = bq.reshape(1, KEY_DIM)
    bk2 = bk.reshape(1, KEY_DIM)

    br1 = _POOL_BLOCK
    q, k = pl.pallas_call(
        _pool_proj_kernel,
        grid=(n // br1,),
        in_specs=[
            pl.BlockSpec((br1, T * IN_DIM), lambda i: (i, 0)),
            pl.BlockSpec((KEY_DIM, IN_DIM), lambda i: (0, 0)),
            pl.BlockSpec((1, KEY_DIM), lambda i: (0, 0)),
            pl.BlockSpec((KEY_DIM, IN_DIM), lambda i: (0, 0)),
            pl.BlockSpec((1, KEY_DIM), lambda i: (0, 0)),
        ],
        out_specs=[
            pl.BlockSpec((br1, KEY_DIM), lambda i: (i, 0)),
            pl.BlockSpec((br1, KEY_DIM), lambda i: (i, 0)),
        ],
        out_shape=[
            jax.ShapeDtypeStruct((n, KEY_DIM), jnp.float32),
            jax.ShapeDtypeStruct((n, KEY_DIM), jnp.float32),
        ],
    )(u2d, Wq, bq2, Wk, bk2)

    br2 = _SCORE_BLOCK
    vals, cols = pl.pallas_call(
        functools.partial(_topk_kernel, n=n, topk=TOPK),
        grid=(n // br2,),
        in_specs=[
            pl.BlockSpec((br2, KEY_DIM), lambda i: (i, 0)),
            pl.BlockSpec((n, KEY_DIM), lambda i: (0, 0)),
        ],
        out_specs=[
            pl.BlockSpec((br2, TOPK), lambda i: (i, 0)),
            pl.BlockSpec((br2, TOPK), lambda i: (i, 0)),
        ],
        out_shape=[
            jax.ShapeDtypeStruct((n, TOPK), jnp.float32),
            jax.ShapeDtypeStruct((n, TOPK), jnp.int32),
        ],
    )(q, k)

    ocols, ovals = _coalesce_sc(cols.reshape(-1), vals.reshape(-1))

    rows = jnp.repeat(jnp.arange(n, dtype=jnp.int32), TOPK)
    indices = jnp.stack([rows, ocols], axis=0)
    return indices, ovals


_NW = 32
_ENT_PER_W = N * TOPK // _NW


def _coalesce_sc_body(cols_hbm, vals_hbm, ocols_hbm, ovals_hbm, cols_v, vals_v):
    wid = jax.lax.axis_index("s") * 2 + jax.lax.axis_index("c")
    base = wid * _ENT_PER_W
    pltpu.sync_copy(cols_hbm.at[pl.ds(base, _ENT_PER_W)], cols_v)
    pltpu.sync_copy(vals_hbm.at[pl.ds(base, _ENT_PER_W)], vals_v)
    offs = jnp.where(jax.lax.iota(jnp.int32, 16) >= TOPK,
                     jnp.int32(16384), jnp.int32(0))

    def body(i, carry):
        k16 = cols_v[pl.ds(i * 16, 16)]
        v16 = vals_v[pl.ds(i * 16, 16)]
        ks, vs = plsc.sort_key_val(k16 + offs, v16, descending=False)
        cols_v[pl.ds(i * 16, 16)] = ks - offs
        vals_v[pl.ds(i * 16, 16)] = vs
        return carry

    jax.lax.fori_loop(0, _ENT_PER_W // 16, body, 0)
    pltpu.sync_copy(cols_v, ocols_hbm.at[pl.ds(base, _ENT_PER_W)])
    pltpu.sync_copy(vals_v, ovals_hbm.at[pl.ds(base, _ENT_PER_W)])


_coalesce_sc = functools.partial(
    pl.kernel,
    mesh=plsc.VectorSubcoreMesh(core_axis_name="c", subcore_axis_name="s"),
    compiler_params=pltpu.CompilerParams(needs_layout_passes=False),
    out_type=[jax.ShapeDtypeStruct((N * TOPK,), jnp.int32),
              jax.ShapeDtypeStruct((N * TOPK,), jnp.float32)],
    scratch_types=[pltpu.VMEM((_ENT_PER_W,), jnp.int32),
                   pltpu.VMEM((_ENT_PER_W,), jnp.float32)],
)(_coalesce_sc_body)
